# Initial kernel scaffold; baseline (speedup 1.0000x reference)
import jax
import jax.numpy as jnp
from jax.experimental import pallas as pl


def kernel(x, batch, W, b):
    raise NotImplementedError("write your pallas kernel here")


# speedup vs baseline: 1.7292x; 1.7292x over previous
import functools

import jax
import jax.numpy as jnp
from jax.experimental import pallas as pl
from jax.experimental.pallas import tpu as pltpu

N = 100000
D = 128
B = 512
R = 1000
NBLK = N // R
NEG_INF = float("-inf")


def _body(x_ref, brow_ref, w1t_ref, w2t_ref, bias_ref, out_ref,
          sums_ref, cnts_ref, maxs_ref):
    i = pl.program_id(0)

    @pl.when(i == 0)
    def _init():
        sums_ref[...] = jnp.zeros((B, D), jnp.float32)
        cnts_ref[...] = jnp.zeros((B, 1), jnp.float32)
        maxs_ref[...] = jnp.full((B, D), NEG_INF, jnp.float32)

    xb = x_ref[...]
    ids_row = brow_ref[0]

    seg_iota = jax.lax.broadcasted_iota(jnp.int32, (B, R), 0)
    onehot = (seg_iota == jnp.broadcast_to(ids_row, (B, R))).astype(jnp.float32)
    sums_ref[...] += jax.lax.dot(
        onehot, xb, precision=jax.lax.Precision.HIGHEST,
        preferred_element_type=jnp.float32)
    cnts_ref[...] += jnp.sum(onehot, axis=1, keepdims=True)

    ids_col = jnp.broadcast_to(ids_row.reshape(R, 1), (R, 1))
    val = xb
    ids = ids_col
    s = 1
    while s < R:
        v_sh = jnp.concatenate(
            [jnp.full((s, D), NEG_INF, jnp.float32), val[:-s, :]], axis=0)
        i_sh = jnp.concatenate(
            [jnp.full((s, 1), -1, jnp.int32), ids[:-s, :]], axis=0)
        val = jnp.where(ids == i_sh, jnp.maximum(val, v_sh), val)
        s *= 2
    ids_next = jnp.concatenate(
        [ids_row[:, 1:], jnp.full((1, 1), -1, jnp.int32)], axis=1)
    is_end = (ids_row != ids_next).astype(jnp.float32)
    endsel = onehot * jnp.broadcast_to(is_end, (B, R))
    blk_max = jax.lax.dot(
        endsel, val, precision=jax.lax.Precision.HIGHEST,
        preferred_element_type=jnp.float32)
    present = jnp.sum(endsel, axis=1, keepdims=True) > 0.0
    maxs_ref[...] = jnp.where(present, jnp.maximum(maxs_ref[...], blk_max),
                              maxs_ref[...])

    @pl.when(i == NBLK - 1)
    def _finish():
        mean = sums_ref[...] / jnp.maximum(cnts_ref[...], 1.0)
        out = (jax.lax.dot(mean, w1t_ref[...],
                           precision=jax.lax.Precision.HIGHEST,
                           preferred_element_type=jnp.float32)
               + jax.lax.dot(maxs_ref[...], w2t_ref[...],
                             precision=jax.lax.Precision.HIGHEST,
                             preferred_element_type=jnp.float32)
               + bias_ref[...])
        out_ref[...] = out


@jax.jit
def kernel(x, batch, W, b):
    batch = batch.astype(jnp.int32)
    brow = batch.reshape(NBLK, 1, R)
    w1t = W[:, :D].T
    w2t = W[:, D:].T
    bias = b.reshape(1, D)
    grid = (NBLK,)
    out = pl.pallas_call(
        _body,
        grid=grid,
        in_specs=[
            pl.BlockSpec((R, D), lambda i: (i, 0)),
            pl.BlockSpec((1, 1, R), lambda i: (i, 0, 0)),
            pl.BlockSpec((D, D), lambda i: (0, 0)),
            pl.BlockSpec((D, D), lambda i: (0, 0)),
            pl.BlockSpec((1, D), lambda i: (0, 0)),
        ],
        out_specs=pl.BlockSpec((B, D), lambda i: (0, 0)),
        out_shape=jax.ShapeDtypeStruct((B, D), jnp.float32),
        scratch_shapes=[
            pltpu.VMEM((B, D), jnp.float32),
            pltpu.VMEM((B, 1), jnp.float32),
            pltpu.VMEM((B, D), jnp.float32),
        ],
        compiler_params=pltpu.CompilerParams(
            dimension_semantics=("arbitrary",)),
    )(x, brow, w1t, w2t, bias)
    return out


# speedup vs baseline: 3.0536x; 1.7659x over previous
import functools

import jax
import jax.numpy as jnp
from jax.experimental import pallas as pl
from jax.experimental.pallas import tpu as pltpu

N = 100000
D = 128
B = 512
R = 1000
NBLK = N // R
NEG_INF = float("-inf")


def _body(x_ref, brow_ref, w1t_ref, w2t_ref, bias_ref, out_ref,
          sums_ref, cnts_ref, maxs_ref):
    i = pl.program_id(0)

    @pl.when(i == 0)
    def _init():
        sums_ref[...] = jnp.zeros((B, D), jnp.float32)
        cnts_ref[...] = jnp.zeros((B, 1), jnp.float32)
        maxs_ref[...] = jnp.full((B, D), NEG_INF, jnp.float32)

    xb = x_ref[...]
    ids_row = brow_ref[0]

    seg_iota = jax.lax.broadcasted_iota(jnp.int32, (B, R), 0)
    onehot = (seg_iota == jnp.broadcast_to(ids_row, (B, R))).astype(jnp.float32)
    sums_ref[...] += jax.lax.dot(
        onehot, xb, preferred_element_type=jnp.float32)
    cnts_ref[...] += jnp.sum(onehot, axis=1, keepdims=True)

    ids_col = jnp.broadcast_to(ids_row.reshape(R, 1), (R, 1))
    val = xb
    ids = ids_col
    s = 1
    while s < R:
        v_sh = jnp.concatenate(
            [jnp.full((s, D), NEG_INF, jnp.float32), val[:-s, :]], axis=0)
        i_sh = jnp.concatenate(
            [jnp.full((s, 1), -1, jnp.int32), ids[:-s, :]], axis=0)
        val = jnp.where(ids == i_sh, jnp.maximum(val, v_sh), val)
        s *= 2
    ids_next = jnp.concatenate(
        [ids_row[:, 1:], jnp.full((1, 1), -1, jnp.int32)], axis=1)
    is_end = (ids_row != ids_next).astype(jnp.float32)
    endsel = onehot * jnp.broadcast_to(is_end, (B, R))
    blk_max = jax.lax.dot(
        endsel, val, preferred_element_type=jnp.float32)
    present = jnp.sum(endsel, axis=1, keepdims=True) > 0.0
    maxs_ref[...] = jnp.where(present, jnp.maximum(maxs_ref[...], blk_max),
                              maxs_ref[...])

    @pl.when(i == NBLK - 1)
    def _finish():
        mean = sums_ref[...] / jnp.maximum(cnts_ref[...], 1.0)
        out = (jax.lax.dot(mean, w1t_ref[...],
                           precision=jax.lax.Precision.HIGHEST,
                           preferred_element_type=jnp.float32)
               + jax.lax.dot(maxs_ref[...], w2t_ref[...],
                             precision=jax.lax.Precision.HIGHEST,
                             preferred_element_type=jnp.float32)
               + bias_ref[...])
        out_ref[...] = out


@jax.jit
def kernel(x, batch, W, b):
    batch = batch.astype(jnp.int32)
    brow = batch.reshape(NBLK, 1, R)
    w1t = W[:, :D].T
    w2t = W[:, D:].T
    bias = b.reshape(1, D)
    grid = (NBLK,)
    out = pl.pallas_call(
        _body,
        grid=grid,
        in_specs=[
            pl.BlockSpec((R, D), lambda i: (i, 0)),
            pl.BlockSpec((1, 1, R), lambda i: (i, 0, 0)),
            pl.BlockSpec((D, D), lambda i: (0, 0)),
            pl.BlockSpec((D, D), lambda i: (0, 0)),
            pl.BlockSpec((1, D), lambda i: (0, 0)),
        ],
        out_specs=pl.BlockSpec((B, D), lambda i: (0, 0)),
        out_shape=jax.ShapeDtypeStruct((B, D), jnp.float32),
        scratch_shapes=[
            pltpu.VMEM((B, D), jnp.float32),
            pltpu.VMEM((B, 1), jnp.float32),
            pltpu.VMEM((B, D), jnp.float32),
        ],
        compiler_params=pltpu.CompilerParams(
            dimension_semantics=("arbitrary",)),
    )(x, brow, w1t, w2t, bias)
    return out
